# baseline (speedup 1.0000x reference)
import jax
import jax.numpy as jnp
from jax.experimental import pallas as pl

N_PHASE = 25000
N_PAD = 25088
BLK = 512


def _mov_mlp_body(x_ref, w1_ref, b1_ref, w2_ref, b2_ref, h_ref):
    x = x_ref[...]
    h = jnp.maximum(
        jnp.dot(x, w1_ref[...], preferred_element_type=jnp.float32) + b1_ref[...], 0.0
    )
    h = jnp.maximum(
        jnp.dot(h, w2_ref[...], preferred_element_type=jnp.float32) + b2_ref[...], 0.0
    )
    h_ref[...] = h


def _head_body(a_ref, wp_ref, bp_ref, w3_ref, b3_ref, w4_ref, b4_ref, o_ref):
    a = a_ref[...]
    p = jnp.maximum(
        jnp.dot(a, wp_ref[...], preferred_element_type=jnp.float32) + bp_ref[...], 0.0
    )
    q = jnp.maximum(
        jnp.dot(p, w3_ref[...], preferred_element_type=jnp.float32) + b3_ref[...], 0.0
    )
    o_ref[...] = jnp.dot(q, w4_ref[...], preferred_element_type=jnp.float32) + b4_ref[...]


def _mov_mlp(x, W1, b1, W2, b2):
    n = x.shape[0]
    grid = n // BLK
    full = lambda s: pl.BlockSpec(s, lambda i: (0,) * len(s))
    return pl.pallas_call(
        _mov_mlp_body,
        grid=(grid,),
        in_specs=[
            pl.BlockSpec((BLK, 128), lambda i: (i, 0)),
            full((128, 512)),
            full((1, 512)),
            full((512, 512)),
            full((1, 512)),
        ],
        out_specs=pl.BlockSpec((BLK, 512), lambda i: (i, 0)),
        out_shape=jax.ShapeDtypeStruct((n, 512), jnp.float32),
    )(x, W1, b1.reshape(1, 512), W2, b2.reshape(1, 512))


def _head(agg, Wp, bp, W3, b3, W4, b4):
    n = agg.shape[0]
    grid = n // BLK
    full = lambda s: pl.BlockSpec(s, lambda i: (0,) * len(s))
    return pl.pallas_call(
        _head_body,
        grid=(grid,),
        in_specs=[
            pl.BlockSpec((BLK, 512), lambda i: (i, 0)),
            full((512, 512)),
            full((1, 512)),
            full((512, 512)),
            full((1, 512)),
            full((512, 1)),
            full((1, 1)),
        ],
        out_specs=pl.BlockSpec((BLK, 1), lambda i: (i, 0)),
        out_shape=jax.ShapeDtypeStruct((n, 1), jnp.float32),
    )(agg, Wp, bp.reshape(1, 512), W3, b3.reshape(1, 512), W4, b4.reshape(1, 1))


def kernel(x_movement, x_phase, edge_index_mp, W1, b1, W2, b2, Wp, bp, W3, b3, W4, b4):
    del x_phase
    xm = jnp.pad(x_movement[:N_PHASE], ((0, N_PAD - N_PHASE), (0, 0)))
    h = _mov_mlp(xm, W1, b1, W2, b2)
    src = edge_index_mp[0]
    dst = edge_index_mp[1]
    agg = jax.ops.segment_max(h[src], dst, num_segments=N_PHASE)
    agg = jnp.where(jnp.isfinite(agg), agg, 0.0)
    agg = jnp.pad(agg, ((0, N_PAD - N_PHASE), (0, 0)))
    out = _head(agg, Wp, bp, W3, b3, W4, b4)
    return out[:N_PHASE]


# speedup vs baseline: 1.5326x; 1.5326x over previous
import dataclasses

import jax
import jax.numpy as jnp
from jax import lax
from jax.experimental import pallas as pl
from jax.experimental.pallas import tpu as pltpu
from jax.experimental.pallas import tpu_sc as plsc

N_PHASE = 25000
N_PAD = 25088
BLK = 512
E = 100000
E_PAD = 100096

NW = 32
PH_PER = 784
AGG_ROWS = 788
SENT = PH_PER
EBLK = 4352
STG_CAP = 2048
FLUSH = 1920
WL_CAP = 102400
ROUND = 64
WFETCH = 128


def _mov_mlp_body(x_ref, w1_ref, b1_ref, w2_ref, b2_ref, h_ref):
    x = x_ref[...]
    h = jnp.maximum(
        jnp.dot(x, w1_ref[...], preferred_element_type=jnp.float32) + b1_ref[...], 0.0
    )
    h = jnp.maximum(
        jnp.dot(h, w2_ref[...], preferred_element_type=jnp.float32) + b2_ref[...], 0.0
    )
    h_ref[0] = h[:, 0:128]
    h_ref[1] = h[:, 128:256]
    h_ref[2] = h[:, 256:384]
    h_ref[3] = h[:, 384:512]


def _mov_mlp(x, W1, b1, W2, b2):
    n = x.shape[0]
    full = lambda s: pl.BlockSpec(s, lambda i: (0,) * len(s))
    return pl.pallas_call(
        _mov_mlp_body,
        grid=(n // BLK,),
        in_specs=[
            pl.BlockSpec((BLK, 128), lambda i: (i, 0)),
            full((128, 512)),
            full((1, 512)),
            full((512, 512)),
            full((1, 512)),
        ],
        out_specs=pl.BlockSpec((4, BLK, 128), lambda i: (0, i, 0)),
        out_shape=jax.ShapeDtypeStruct((4, n, 128), jnp.float32),
    )(x, W1, b1.reshape(1, 512), W2, b2.reshape(1, 512))


def _head_body(a_ref, wp_ref, bp_ref, w3_ref, b3_ref, w4_ref, b4_ref, o_ref):
    acc = jnp.dot(a_ref[0], wp_ref[0:128, :], preferred_element_type=jnp.float32)
    acc += jnp.dot(a_ref[1], wp_ref[128:256, :], preferred_element_type=jnp.float32)
    acc += jnp.dot(a_ref[2], wp_ref[256:384, :], preferred_element_type=jnp.float32)
    acc += jnp.dot(a_ref[3], wp_ref[384:512, :], preferred_element_type=jnp.float32)
    p = jnp.maximum(acc + bp_ref[...], 0.0)
    q = jnp.maximum(
        jnp.dot(p, w3_ref[...], preferred_element_type=jnp.float32) + b3_ref[...], 0.0
    )
    o_ref[...] = jnp.dot(q, w4_ref[...], preferred_element_type=jnp.float32) + b4_ref[...]


def _head(a, Wp, bp, W3, b3, W4, b4):
    n = a.shape[1]
    full = lambda s: pl.BlockSpec(s, lambda i: (0,) * len(s))
    return pl.pallas_call(
        _head_body,
        grid=(n // BLK,),
        in_specs=[
            pl.BlockSpec((4, BLK, 128), lambda i: (0, i, 0)),
            full((512, 512)),
            full((1, 512)),
            full((512, 512)),
            full((1, 512)),
            full((512, 1)),
            full((1, 1)),
        ],
        out_specs=pl.BlockSpec((BLK, 1), lambda i: (i, 0)),
        out_shape=jax.ShapeDtypeStruct((n, 1), jnp.float32),
    )(a, Wp, bp.reshape(1, 512), W3, b3.reshape(1, 512), W4, b4.reshape(1, 1))


def _seg_max_body(src_hbm, dst_hbm, h_hbm,
                  a_hbm, wl_hbm,
                  agg_t, stg, esrc, edst, wlv, gbuf,
                  wsem0, wsem1, gsem0, gsem1):
    wid = lax.axis_index("s") * 2 + lax.axis_index("c")
    iota16 = lax.iota(jnp.int32, 16)
    my_base = wid * PH_PER
    wl_base = wid * WL_CAP
    wsems = (wsem0, wsem1)
    gsems = (gsem0, gsem1)

    def chunk_body(k, carry):
        ptr, wtot = carry
        s16 = esrc[pl.ds(k * 16, 16)]
        d16 = edst[pl.ds(k * 16, 16)]
        u = d16 - my_base
        mask = (u >= 0) & (u < PH_PER)
        mi = jnp.where(mask, 1, 0).astype(jnp.int32)
        pos = ptr + plsc.cumsum(mi) - 1
        packed = lax.shift_left(s16, 10) | u
        plsc.store_scatter(stg, [pos], packed, mask=mask)
        ptr = ptr + jnp.sum(mi)

        def do_flush(p, w):
            pltpu.sync_copy(stg.at[pl.ds(0, FLUSH)],
                            wl_hbm.at[pl.ds(pl.multiple_of(wl_base + w, 128), FLUSH)])
            rem = stg[pl.ds(FLUSH, 16)]
            stg[pl.ds(0, 16)] = rem
            return p - FLUSH, w + FLUSH

        ptr, wtot = lax.cond(ptr >= FLUSH, do_flush, lambda p, w: (p, w), ptr, wtot)
        return ptr, wtot

    def block_body(b, carry):
        pltpu.sync_copy(src_hbm.at[pl.ds(pl.multiple_of(b * EBLK, 128), EBLK)], esrc)
        pltpu.sync_copy(dst_hbm.at[pl.ds(pl.multiple_of(b * EBLK, 128), EBLK)], edst)
        return pl.loop(0, EBLK // 16, init_carry=carry)(chunk_body)

    ptr, wtot = pl.loop(0, E_PAD // EBLK,
                        init_carry=(jnp.int32(0), jnp.int32(0)))(block_body)

    sent_v = jnp.full((16,), SENT, jnp.int32)
    for t in range(ROUND // 16):
        plsc.store_scatter(stg, [ptr + iota16 + 16 * t], sent_v)
    ptrp = lax.div(ptr + (ROUND - 1), ROUND) * ROUND
    pltpu.sync_copy(stg.at[pl.ds(0, STG_CAP)],
                    wl_hbm.at[pl.ds(pl.multiple_of(wl_base + wtot, 128), STG_CAP)])
    n_w = wtot + ptrp
    n_rounds = lax.div(n_w, ROUND)
    n_super = lax.div(n_rounds + 1, 2)

    def issue_gathers(f, wlb, half, gb):
        for g in range(4):
            pk = wlv[wlb, pl.ds(half * 64 + g * 16, 16)]
            s16 = lax.shift_right_logical(pk, 10) + f * N_PAD
            pltpu.async_copy(h_hbm.at[s16], gbuf.at[gb, g], gsems[gb])

    def drain_gathers(gb):
        for g in range(4):
            pltpu.make_async_copy(
                h_hbm.at[pl.ds(0, 16)], gbuf.at[gb, g], gsems[gb]
            ).wait()

    def process_round(wlb, half, gb):
        for g in range(4):
            pk = wlv[wlb, pl.ds(half * 64 + g * 16, 16)]
            u16 = pk & 1023

            @pl.loop(0, 16)
            def _(e):
                esplat = jnp.full((16,), 0, jnp.int32) + e
                drow = lax.gather(
                    u16, esplat[:, None],
                    lax.GatherDimensionNumbers(
                        offset_dims=(), collapsed_slice_dims=(0,),
                        start_index_map=(0,)),
                    slice_sizes=(1,),
                    mode=lax.GatherScatterMode.PROMISE_IN_BOUNDS)
                for v in range(8):
                    fidx = iota16 + 16 * v
                    gval = gbuf[gb, g, e, pl.ds(16 * v, 16)]
                    old = plsc.load_gather(agg_t, [drow, fidx])
                    plsc.store_scatter(agg_t, [drow, fidx], jnp.maximum(old, gval))

    def fetch_wl(j, wlb):
        pltpu.async_copy(
            wl_hbm.at[pl.ds(pl.multiple_of(wl_base + j * WFETCH, 128), WFETCH)],
            wlv.at[wlb], wsems[wlb])

    def wait_wl(wlb):
        pltpu.make_async_copy(wl_hbm.at[pl.ds(0, WFETCH)], wlv.at[wlb],
                              wsems[wlb]).wait()

    z16 = jnp.zeros((16,), jnp.float32)

    @pl.loop(0, 4)
    def _(f):
        @pl.loop(0, AGG_ROWS)
        def _(r):
            for c in range(8):
                agg_t[r, pl.ds(c * 16, 16)] = z16

        @pl.when(n_rounds > 0)
        def _():
            pltpu.sync_copy(
                wl_hbm.at[pl.ds(pl.multiple_of(wl_base, 128), WFETCH)], wlv.at[0])
            issue_gathers(f, 0, 0, 0)

        @pl.loop(0, n_super)
        def _(j):
            wlb = lax.rem(j, 2)
            r0 = j * 2

            def super_body(wlb, wlbn):
                @pl.when(j + 1 < n_super)
                def _():
                    fetch_wl(j + 1, wlbn)

                @pl.when(r0 + 1 < n_rounds)
                def _():
                    issue_gathers(f, wlb, 1, 1)

                drain_gathers(0)
                process_round(wlb, 0, 0)

                @pl.when(r0 + 1 < n_rounds)
                def _():
                    @pl.when(j + 1 < n_super)
                    def _():
                        wait_wl(wlbn)
                        issue_gathers(f, wlbn, 0, 0)

                    drain_gathers(1)
                    process_round(wlb, 1, 1)

            @pl.when(wlb == 0)
            def _():
                super_body(0, 1)

            @pl.when(wlb == 1)
            def _():
                super_body(1, 0)

        pltpu.sync_copy(
            agg_t.at[pl.ds(0, PH_PER)],
            a_hbm.at[pl.ds(pl.multiple_of(f * N_PAD + my_base, 8), PH_PER)])


def _seg_max(src, dst, h):
    mesh = plsc.VectorSubcoreMesh(core_axis_name="c", subcore_axis_name="s")
    cp = pltpu.CompilerParams()
    if "needs_layout_passes" in pltpu.CompilerParams.__dataclass_fields__:
        cp = dataclasses.replace(cp, needs_layout_passes=False)
    outs = pl.kernel(
        _seg_max_body,
        out_type=[jax.ShapeDtypeStruct((4 * N_PAD, 128), jnp.float32),
                  jax.ShapeDtypeStruct((NW * WL_CAP,), jnp.int32)],
        mesh=mesh,
        scratch_types=[
            pltpu.VMEM((AGG_ROWS, 128), jnp.float32),
            pltpu.VMEM((STG_CAP,), jnp.int32),
            pltpu.VMEM((EBLK,), jnp.int32),
            pltpu.VMEM((EBLK,), jnp.int32),
            pltpu.VMEM((2, WFETCH), jnp.int32),
            pltpu.VMEM((2, 4, 16, 128), jnp.float32),
            pltpu.SemaphoreType.DMA,
            pltpu.SemaphoreType.DMA,
            pltpu.SemaphoreType.DMA,
            pltpu.SemaphoreType.DMA,
        ],
        compiler_params=cp,
    )(src, dst, h)
    return outs[0]


def kernel(x_movement, x_phase, edge_index_mp, W1, b1, W2, b2, Wp, bp, W3, b3, W4, b4):
    del x_phase
    xm = jnp.pad(x_movement[:N_PHASE], ((0, N_PAD - N_PHASE), (0, 0)))
    h = _mov_mlp(xm, W1, b1, W2, b2).reshape(4 * N_PAD, 128)
    src = jnp.pad(edge_index_mp[0], (0, E_PAD - E))
    dst = jnp.pad(edge_index_mp[1], (0, E_PAD - E), constant_values=1 << 20)
    a = _seg_max(src, dst, h).reshape(4, N_PAD, 128)
    out = _head(a, Wp, bp, W3, b3, W4, b4)
    return out[:N_PHASE]


# speedup vs baseline: 1.6329x; 1.0654x over previous
import dataclasses

import jax
import jax.numpy as jnp
from jax import lax
from jax.experimental import pallas as pl
from jax.experimental.pallas import tpu as pltpu
from jax.experimental.pallas import tpu_sc as plsc

N_PHASE = 25000
N_PAD = 25088
BLK = 512
E = 100000
E_PAD = 100096

NW = 32
PH_PER = 784
AGG_ROWS = 788
SENT = PH_PER
EBLK = 4352
STG_CAP = 2048
FLUSH = 1920
WL_CAP = 102400
ROUND = 64
WFETCH = 128


def _mov_mlp_body(x_ref, w1_ref, b1_ref, w2_ref, b2_ref, h_ref):
    x = x_ref[...]
    h = jnp.maximum(
        jnp.dot(x, w1_ref[...], preferred_element_type=jnp.float32) + b1_ref[...], 0.0
    )
    h = jnp.maximum(
        jnp.dot(h, w2_ref[...], preferred_element_type=jnp.float32) + b2_ref[...], 0.0
    )
    h_ref[0] = h[:, 0:128]
    h_ref[1] = h[:, 128:256]
    h_ref[2] = h[:, 256:384]
    h_ref[3] = h[:, 384:512]


def _mov_mlp(x, W1, b1, W2, b2):
    n = x.shape[0]
    full = lambda s: pl.BlockSpec(s, lambda i: (0,) * len(s))
    return pl.pallas_call(
        _mov_mlp_body,
        grid=(n // BLK,),
        in_specs=[
            pl.BlockSpec((BLK, 128), lambda i: (i, 0)),
            full((128, 512)),
            full((1, 512)),
            full((512, 512)),
            full((1, 512)),
        ],
        out_specs=pl.BlockSpec((4, BLK, 128), lambda i: (0, i, 0)),
        out_shape=jax.ShapeDtypeStruct((4, n, 128), jnp.float32),
    )(x, W1, b1.reshape(1, 512), W2, b2.reshape(1, 512))


def _head_body(a_ref, wp_ref, bp_ref, w3_ref, b3_ref, w4_ref, b4_ref, o_ref):
    acc = jnp.dot(a_ref[0], wp_ref[0:128, :], preferred_element_type=jnp.float32)
    acc += jnp.dot(a_ref[1], wp_ref[128:256, :], preferred_element_type=jnp.float32)
    acc += jnp.dot(a_ref[2], wp_ref[256:384, :], preferred_element_type=jnp.float32)
    acc += jnp.dot(a_ref[3], wp_ref[384:512, :], preferred_element_type=jnp.float32)
    p = jnp.maximum(acc + bp_ref[...], 0.0)
    q = jnp.maximum(
        jnp.dot(p, w3_ref[...], preferred_element_type=jnp.float32) + b3_ref[...], 0.0
    )
    o_ref[...] = jnp.dot(q, w4_ref[...], preferred_element_type=jnp.float32) + b4_ref[...]


def _head(a, Wp, bp, W3, b3, W4, b4):
    n = a.shape[1]
    full = lambda s: pl.BlockSpec(s, lambda i: (0,) * len(s))
    return pl.pallas_call(
        _head_body,
        grid=(n // BLK,),
        in_specs=[
            pl.BlockSpec((4, BLK, 128), lambda i: (0, i, 0)),
            full((512, 512)),
            full((1, 512)),
            full((512, 512)),
            full((1, 512)),
            full((512, 1)),
            full((1, 1)),
        ],
        out_specs=pl.BlockSpec((BLK, 1), lambda i: (i, 0)),
        out_shape=jax.ShapeDtypeStruct((n, 1), jnp.float32),
    )(a, Wp, bp.reshape(1, 512), W3, b3.reshape(1, 512), W4, b4.reshape(1, 1))


def _seg_max_body(src_hbm, dst_hbm, h_hbm,
                  a_hbm, wl_hbm,
                  agg_t, stg, esrc, edst, wlv, gbuf,
                  wsem0, wsem1, gsem0, gsem1):
    wid = lax.axis_index("s") * 2 + lax.axis_index("c")
    iota16 = lax.iota(jnp.int32, 16)
    my_base = wid * PH_PER
    wl_base = wid * WL_CAP
    wsems = (wsem0, wsem1)
    gsems = (gsem0, gsem1)

    def chunk_body(k, carry):
        ptr, wtot = carry
        s16 = esrc[pl.ds(k * 16, 16)]
        d16 = edst[pl.ds(k * 16, 16)]
        u = d16 - my_base
        mask = (u >= 0) & (u < PH_PER)
        mi = jnp.where(mask, 1, 0).astype(jnp.int32)
        pos = ptr + plsc.cumsum(mi) - 1
        packed = lax.shift_left(s16, 10) | u
        plsc.store_scatter(stg, [pos], packed, mask=mask)
        ptr = ptr + jnp.sum(mi)

        def do_flush(p, w):
            pltpu.sync_copy(stg.at[pl.ds(0, FLUSH)],
                            wl_hbm.at[pl.ds(pl.multiple_of(wl_base + w, 128), FLUSH)])
            rem = stg[pl.ds(FLUSH, 16)]
            stg[pl.ds(0, 16)] = rem
            return p - FLUSH, w + FLUSH

        ptr, wtot = lax.cond(ptr >= FLUSH, do_flush, lambda p, w: (p, w), ptr, wtot)
        return ptr, wtot

    def block_body(b, carry):
        pltpu.sync_copy(src_hbm.at[pl.ds(pl.multiple_of(b * EBLK, 128), EBLK)], esrc)
        pltpu.sync_copy(dst_hbm.at[pl.ds(pl.multiple_of(b * EBLK, 128), EBLK)], edst)
        return pl.loop(0, EBLK // 16, init_carry=carry)(chunk_body)

    ptr, wtot = pl.loop(0, E_PAD // EBLK,
                        init_carry=(jnp.int32(0), jnp.int32(0)))(block_body)

    sent_v = jnp.full((16,), SENT, jnp.int32)
    for t in range(ROUND // 16):
        plsc.store_scatter(stg, [ptr + iota16 + 16 * t], sent_v)
    ptrp = lax.div(ptr + (ROUND - 1), ROUND) * ROUND
    pltpu.sync_copy(stg.at[pl.ds(0, STG_CAP)],
                    wl_hbm.at[pl.ds(pl.multiple_of(wl_base + wtot, 128), STG_CAP)])
    n_w = wtot + ptrp
    n_rounds = lax.div(n_w, ROUND)
    n_super = lax.div(n_rounds + 1, 2)

    def issue_gathers(f, wlb, half, gb):
        for g in range(4):
            pk = wlv[wlb, pl.ds(half * 64 + g * 16, 16)]
            s16 = lax.shift_right_logical(pk, 10) + f * N_PAD
            pltpu.async_copy(h_hbm.at[s16], gbuf.at[gb, pl.ds(g * 16, 16)],
                             gsems[gb])

    def drain_gathers(gb):
        for g in range(4):
            pltpu.make_async_copy(
                h_hbm.at[pl.ds(0, 16)], gbuf.at[gb, pl.ds(g * 16, 16)],
                gsems[gb]
            ).wait()

    def process_round(wlb, half, gb):
        @pl.loop(0, 4)
        def _(g):
            pk16 = wlv[wlb, pl.ds(half * 64 + g * 16, 16)]
            u16 = pk16 & 1023
            for e in range(16):
                u = u16[e]
                row = g * 16 + e
                for v in range(8):
                    gval = gbuf[gb, row, pl.ds(16 * v, 16)]
                    old = agg_t[u, pl.ds(16 * v, 16)]
                    agg_t[u, pl.ds(16 * v, 16)] = jnp.maximum(old, gval)

    def fetch_wl(j, wlb):
        off = pl.ds(pl.multiple_of(wl_base + j * WFETCH, 128), WFETCH)
        pltpu.async_copy(wl_hbm.at[off], wlv.at[wlb], wsems[wlb])

    def wait_wl(wlb):
        pltpu.make_async_copy(wl_hbm.at[pl.ds(0, WFETCH)], wlv.at[wlb],
                              wsems[wlb]).wait()

    z16 = jnp.zeros((16,), jnp.float32)

    @pl.loop(0, 4)
    def _(f):
        @pl.loop(0, AGG_ROWS)
        def _(r):
            for c in range(8):
                agg_t[r, pl.ds(c * 16, 16)] = z16

        @pl.when(n_rounds > 0)
        def _():
            pltpu.sync_copy(
                wl_hbm.at[pl.ds(pl.multiple_of(wl_base, 128), WFETCH)], wlv.at[0])
            issue_gathers(f, 0, 0, 0)

        @pl.loop(0, n_super)
        def _(j):
            wlb = lax.rem(j, 2)
            r0 = j * 2

            def super_body(wlb, wlbn):
                @pl.when(j + 1 < n_super)
                def _():
                    fetch_wl(j + 1, wlbn)

                @pl.when(r0 + 1 < n_rounds)
                def _():
                    issue_gathers(f, wlb, 1, 1)

                drain_gathers(0)
                process_round(wlb, 0, 0)

                @pl.when(r0 + 1 < n_rounds)
                def _():
                    @pl.when(j + 1 < n_super)
                    def _():
                        wait_wl(wlbn)
                        issue_gathers(f, wlbn, 0, 0)

                    drain_gathers(1)
                    process_round(wlb, 1, 1)

            @pl.when(wlb == 0)
            def _():
                super_body(0, 1)

            @pl.when(wlb == 1)
            def _():
                super_body(1, 0)

        pltpu.sync_copy(
            agg_t.at[pl.ds(0, PH_PER)],
            a_hbm.at[pl.ds(pl.multiple_of(f * N_PAD + my_base, 8), PH_PER)])


def _seg_max(src, dst, h):
    mesh = plsc.VectorSubcoreMesh(core_axis_name="c", subcore_axis_name="s")
    cp = pltpu.CompilerParams()
    if "needs_layout_passes" in pltpu.CompilerParams.__dataclass_fields__:
        cp = dataclasses.replace(cp, needs_layout_passes=False)
    outs = pl.kernel(
        _seg_max_body,
        out_type=[jax.ShapeDtypeStruct((4 * N_PAD, 128), jnp.float32),
                  jax.ShapeDtypeStruct((NW * WL_CAP,), jnp.int32)],
        mesh=mesh,
        scratch_types=[
            pltpu.VMEM((AGG_ROWS, 128), jnp.float32),
            pltpu.VMEM((STG_CAP,), jnp.int32),
            pltpu.VMEM((EBLK,), jnp.int32),
            pltpu.VMEM((EBLK,), jnp.int32),
            pltpu.VMEM((2, WFETCH), jnp.int32),
            pltpu.VMEM((2, 64, 128), jnp.float32),
            pltpu.SemaphoreType.DMA,
            pltpu.SemaphoreType.DMA,
            pltpu.SemaphoreType.DMA,
            pltpu.SemaphoreType.DMA,
        ],
        compiler_params=cp,
    )(src, dst, h)
    return outs[0]


def kernel(x_movement, x_phase, edge_index_mp, W1, b1, W2, b2, Wp, bp, W3, b3, W4, b4):
    del x_phase
    xm = jnp.pad(x_movement[:N_PHASE], ((0, N_PAD - N_PHASE), (0, 0)))
    h = _mov_mlp(xm, W1, b1, W2, b2).reshape(4 * N_PAD, 128)
    src = jnp.pad(edge_index_mp[0], (0, E_PAD - E))
    dst = jnp.pad(edge_index_mp[1], (0, E_PAD - E), constant_values=1 << 20)
    a = _seg_max(src, dst, h).reshape(4, N_PAD, 128)
    out = _head(a, Wp, bp, W3, b3, W4, b4)
    return out[:N_PHASE]


# speedup vs baseline: 1.7147x; 1.0501x over previous
import dataclasses

import jax
import jax.numpy as jnp
from jax import lax
from jax.experimental import pallas as pl
from jax.experimental.pallas import tpu as pltpu
from jax.experimental.pallas import tpu_sc as plsc

N_PHASE = 25000
N_PAD = 25088
BLK = 512
E = 100000
E_PAD = 100096

NW = 32
PH_PER = 784
AGG_ROWS = 788
SENT = PH_PER
EBLK = 4352
STG_CAP = 2048
FLUSH = 1920
WL_CAP = 102400
ROUND = 64
WFETCH = 128


def _mov_mlp_body(x_ref, w1_ref, b1_ref, w2_ref, b2_ref, h_ref):
    x = x_ref[...]
    h = jnp.maximum(
        jnp.dot(x, w1_ref[...], preferred_element_type=jnp.float32) + b1_ref[...], 0.0
    )
    h = jnp.maximum(
        jnp.dot(h, w2_ref[...], preferred_element_type=jnp.float32) + b2_ref[...], 0.0
    )
    h_ref[0] = h[:, 0:128]
    h_ref[1] = h[:, 128:256]
    h_ref[2] = h[:, 256:384]
    h_ref[3] = h[:, 384:512]


def _mov_mlp(x, W1, b1, W2, b2):
    n = x.shape[0]
    full = lambda s: pl.BlockSpec(s, lambda i: (0,) * len(s))
    return pl.pallas_call(
        _mov_mlp_body,
        grid=(n // BLK,),
        in_specs=[
            pl.BlockSpec((BLK, 128), lambda i: (i, 0)),
            full((128, 512)),
            full((1, 512)),
            full((512, 512)),
            full((1, 512)),
        ],
        out_specs=pl.BlockSpec((4, BLK, 128), lambda i: (0, i, 0)),
        out_shape=jax.ShapeDtypeStruct((4, n, 128), jnp.float32),
    )(x, W1, b1.reshape(1, 512), W2, b2.reshape(1, 512))


def _head_body(a_ref, wp_ref, bp_ref, w3_ref, b3_ref, w4_ref, b4_ref, o_ref):
    acc = jnp.dot(a_ref[0], wp_ref[0:128, :], preferred_element_type=jnp.float32)
    acc += jnp.dot(a_ref[1], wp_ref[128:256, :], preferred_element_type=jnp.float32)
    acc += jnp.dot(a_ref[2], wp_ref[256:384, :], preferred_element_type=jnp.float32)
    acc += jnp.dot(a_ref[3], wp_ref[384:512, :], preferred_element_type=jnp.float32)
    p = jnp.maximum(acc + bp_ref[...], 0.0)
    q = jnp.maximum(
        jnp.dot(p, w3_ref[...], preferred_element_type=jnp.float32) + b3_ref[...], 0.0
    )
    o_ref[...] = jnp.dot(q, w4_ref[...], preferred_element_type=jnp.float32) + b4_ref[...]


def _head(a, Wp, bp, W3, b3, W4, b4):
    n = a.shape[1]
    full = lambda s: pl.BlockSpec(s, lambda i: (0,) * len(s))
    return pl.pallas_call(
        _head_body,
        grid=(n // BLK,),
        in_specs=[
            pl.BlockSpec((4, BLK, 128), lambda i: (0, i, 0)),
            full((512, 512)),
            full((1, 512)),
            full((512, 512)),
            full((1, 512)),
            full((512, 1)),
            full((1, 1)),
        ],
        out_specs=pl.BlockSpec((BLK, 1), lambda i: (i, 0)),
        out_shape=jax.ShapeDtypeStruct((n, 1), jnp.float32),
    )(a, Wp, bp.reshape(1, 512), W3, b3.reshape(1, 512), W4, b4.reshape(1, 1))


def _compact_body(src_hbm, dst_hbm, wl_hbm, stg, esrc, edst):
    wid = lax.axis_index("s") * 2 + lax.axis_index("c")
    iota16 = lax.iota(jnp.int32, 16)
    my_base = wid * PH_PER
    wl_base = wid * WL_CAP

    def chunk_body(k, carry):
        ptr, wtot = carry
        s16 = esrc[pl.ds(k * 16, 16)]
        d16 = edst[pl.ds(k * 16, 16)]
        u = d16 - my_base
        mask = (u >= 0) & (u < PH_PER)
        mi = jnp.where(mask, 1, 0).astype(jnp.int32)
        pos = ptr + plsc.cumsum(mi) - 1
        packed = lax.shift_left(s16, 10) | u
        plsc.store_scatter(stg, [pos], packed, mask=mask)
        ptr = ptr + jnp.sum(mi)

        def do_flush(p, w):
            pltpu.sync_copy(
                stg.at[pl.ds(0, FLUSH)],
                wl_hbm.at[pl.ds(pl.multiple_of(wl_base + 128 + w, 128), FLUSH)])
            rem = stg[pl.ds(FLUSH, 16)]
            stg[pl.ds(0, 16)] = rem
            return p - FLUSH, w + FLUSH

        ptr, wtot = lax.cond(ptr >= FLUSH, do_flush, lambda p, w: (p, w), ptr, wtot)
        return ptr, wtot

    def block_body(b, carry):
        pltpu.sync_copy(src_hbm.at[pl.ds(pl.multiple_of(b * EBLK, 128), EBLK)], esrc)
        pltpu.sync_copy(dst_hbm.at[pl.ds(pl.multiple_of(b * EBLK, 128), EBLK)], edst)
        return pl.loop(0, EBLK // 16, init_carry=carry)(chunk_body)

    ptr, wtot = pl.loop(0, E_PAD // EBLK,
                        init_carry=(jnp.int32(0), jnp.int32(0)))(block_body)

    sent_v = jnp.full((16,), SENT, jnp.int32)
    for t in range(ROUND // 16):
        plsc.store_scatter(stg, [ptr + iota16 + 16 * t], sent_v)
    ptrp = lax.div(ptr + (ROUND - 1), ROUND) * ROUND
    pltpu.sync_copy(
        stg.at[pl.ds(0, STG_CAP)],
        wl_hbm.at[pl.ds(pl.multiple_of(wl_base + 128 + wtot, 128), STG_CAP)])
    stg[pl.ds(0, 16)] = jnp.full((16,), 0, jnp.int32) + (wtot + ptrp)
    pltpu.sync_copy(stg.at[pl.ds(0, 128)],
                    wl_hbm.at[pl.ds(pl.multiple_of(wl_base, 128), 128)])


def _compact(src, dst):
    mesh = plsc.VectorSubcoreMesh(core_axis_name="c", subcore_axis_name="s")
    cp = pltpu.CompilerParams()
    if "needs_layout_passes" in pltpu.CompilerParams.__dataclass_fields__:
        cp = dataclasses.replace(cp, needs_layout_passes=False)
    return pl.kernel(
        _compact_body,
        out_type=[jax.ShapeDtypeStruct((NW * WL_CAP,), jnp.int32)],
        mesh=mesh,
        scratch_types=[
            pltpu.VMEM((STG_CAP,), jnp.int32),
            pltpu.VMEM((EBLK,), jnp.int32),
            pltpu.VMEM((EBLK,), jnp.int32),
        ],
        compiler_params=cp,
    )(src, dst)[0]


def _gmax_body(wl_hbm, h_hbm, a_hbm,
               agg_t, wlv, gbuf,
               wsem0, wsem1, gsem0, gsem1):
    wid = lax.axis_index("s") * 2 + lax.axis_index("c")
    my_base = wid * PH_PER
    wl_base = wid * WL_CAP
    wsems = (wsem0, wsem1)
    gsems = (gsem0, gsem1)

    pltpu.sync_copy(wl_hbm.at[pl.ds(pl.multiple_of(wl_base, 128), 128)],
                    wlv.at[0])
    n_w = wlv[0, pl.ds(0, 16)][0]
    n_rounds = lax.div(n_w, ROUND)
    n_super = lax.div(n_rounds + 1, 2)

    def issue_gathers(f, wlb, half, gb):
        for g in range(4):
            pk = wlv[wlb, pl.ds(half * 64 + g * 16, 16)]
            s16 = lax.shift_right_logical(pk, 10) + f * N_PAD
            pltpu.async_copy(h_hbm.at[s16], gbuf.at[gb, pl.ds(g * 16, 16)],
                             gsems[gb])

    def drain_gathers(gb):
        for g in range(4):
            pltpu.make_async_copy(
                h_hbm.at[pl.ds(0, 16)], gbuf.at[gb, pl.ds(g * 16, 16)],
                gsems[gb]
            ).wait()

    def process_round(wlb, half, gb):
        @pl.loop(0, 4)
        def _(g):
            pk16 = wlv[wlb, pl.ds(half * 64 + g * 16, 16)]
            u16 = pk16 & 1023
            for e in range(16):
                u = u16[e]
                row = g * 16 + e
                for v in range(8):
                    gval = gbuf[gb, row, pl.ds(16 * v, 16)]
                    old = agg_t[u, pl.ds(16 * v, 16)]
                    agg_t[u, pl.ds(16 * v, 16)] = jnp.maximum(old, gval)

    def fetch_wl(j, wlb):
        off = pl.ds(pl.multiple_of(wl_base + 128 + j * WFETCH, 128), WFETCH)
        pltpu.async_copy(wl_hbm.at[off], wlv.at[wlb], wsems[wlb])

    def wait_wl(wlb):
        pltpu.make_async_copy(wl_hbm.at[pl.ds(0, WFETCH)], wlv.at[wlb],
                              wsems[wlb]).wait()

    z16 = jnp.zeros((16,), jnp.float32)

    @pl.loop(0, 4)
    def _(f):
        @pl.loop(0, AGG_ROWS)
        def _(r):
            for c in range(8):
                agg_t[r, pl.ds(c * 16, 16)] = z16

        @pl.when(n_rounds > 0)
        def _():
            pltpu.sync_copy(
                wl_hbm.at[pl.ds(pl.multiple_of(wl_base + 128, 128), WFETCH)],
                wlv.at[0])
            issue_gathers(f, 0, 0, 0)

        @pl.loop(0, n_super)
        def _(j):
            wlb = lax.rem(j, 2)
            r0 = j * 2

            def super_body(wlb, wlbn):
                @pl.when(j + 1 < n_super)
                def _():
                    fetch_wl(j + 1, wlbn)

                @pl.when(r0 + 1 < n_rounds)
                def _():
                    issue_gathers(f, wlb, 1, 1)

                drain_gathers(0)
                process_round(wlb, 0, 0)

                @pl.when(r0 + 1 < n_rounds)
                def _():
                    @pl.when(j + 1 < n_super)
                    def _():
                        wait_wl(wlbn)
                        issue_gathers(f, wlbn, 0, 0)

                    drain_gathers(1)
                    process_round(wlb, 1, 1)

            @pl.when(wlb == 0)
            def _():
                super_body(0, 1)

            @pl.when(wlb == 1)
            def _():
                super_body(1, 0)

        pltpu.sync_copy(
            agg_t.at[pl.ds(0, PH_PER)],
            a_hbm.at[pl.ds(pl.multiple_of(f * N_PAD + my_base, 8), PH_PER)])


def _gmax(wl, h):
    mesh = plsc.VectorSubcoreMesh(core_axis_name="c", subcore_axis_name="s")
    cp = pltpu.CompilerParams()
    if "needs_layout_passes" in pltpu.CompilerParams.__dataclass_fields__:
        cp = dataclasses.replace(cp, needs_layout_passes=False)
    return pl.kernel(
        _gmax_body,
        out_type=[jax.ShapeDtypeStruct((4 * N_PAD, 128), jnp.float32)],
        mesh=mesh,
        scratch_types=[
            pltpu.VMEM((AGG_ROWS, 128), jnp.float32),
            pltpu.VMEM((2, WFETCH), jnp.int32),
            pltpu.VMEM((2, 64, 128), jnp.float32),
            pltpu.SemaphoreType.DMA,
            pltpu.SemaphoreType.DMA,
            pltpu.SemaphoreType.DMA,
            pltpu.SemaphoreType.DMA,
        ],
        compiler_params=cp,
    )(wl, h)[0]


def kernel(x_movement, x_phase, edge_index_mp, W1, b1, W2, b2, Wp, bp, W3, b3, W4, b4):
    del x_phase
    src = jnp.pad(edge_index_mp[0], (0, E_PAD - E))
    dst = jnp.pad(edge_index_mp[1], (0, E_PAD - E), constant_values=1 << 20)
    wl = _compact(src, dst)
    xm = jnp.pad(x_movement[:N_PHASE], ((0, N_PAD - N_PHASE), (0, 0)))
    h = _mov_mlp(xm, W1, b1, W2, b2).reshape(4 * N_PAD, 128)
    a = _gmax(wl, h).reshape(4, N_PAD, 128)
    out = _head(a, Wp, bp, W3, b3, W4, b4)
    return out[:N_PHASE]


# speedup vs baseline: 1.7249x; 1.0060x over previous
import dataclasses

import jax
import jax.numpy as jnp
from jax import lax
from jax.experimental import pallas as pl
from jax.experimental.pallas import tpu as pltpu
from jax.experimental.pallas import tpu_sc as plsc

N_PHASE = 25000
N_PAD = 25088
BLK = 512
E = 100000
E_PAD = 100096

NW = 32
PH_PER = 784
AGG_ROWS = 788
SENT = PH_PER
EBLK = 4352
STG_CAP = 2048
FLUSH = 1920
WL_CAP = 102400
ROUND = 64
WFETCH = 128


def _mov_mlp_body(x_ref, w1_ref, b1_ref, w2_ref, b2_ref, h_ref):
    x = x_ref[...]
    h = jnp.maximum(
        jnp.dot(x, w1_ref[...], preferred_element_type=jnp.float32) + b1_ref[...], 0.0
    )
    h = jnp.maximum(
        jnp.dot(h, w2_ref[...], preferred_element_type=jnp.float32) + b2_ref[...], 0.0
    )
    h_ref[0] = h[:, 0:128]
    h_ref[1] = h[:, 128:256]
    h_ref[2] = h[:, 256:384]
    h_ref[3] = h[:, 384:512]


def _mov_mlp(x, W1, b1, W2, b2):
    n = x.shape[0]
    full = lambda s: pl.BlockSpec(s, lambda i: (0,) * len(s))
    return pl.pallas_call(
        _mov_mlp_body,
        grid=(n // BLK,),
        in_specs=[
            pl.BlockSpec((BLK, 128), lambda i: (i, 0)),
            full((128, 512)),
            full((1, 512)),
            full((512, 512)),
            full((1, 512)),
        ],
        out_specs=pl.BlockSpec((4, BLK, 128), lambda i: (0, i, 0)),
        out_shape=jax.ShapeDtypeStruct((4, n, 128), jnp.float32),
    )(x, W1, b1.reshape(1, 512), W2, b2.reshape(1, 512))


def _head_body(a_ref, wp_ref, bp_ref, w3_ref, b3_ref, w4_ref, b4_ref, o_ref):
    acc = jnp.dot(a_ref[0], wp_ref[0:128, :], preferred_element_type=jnp.float32)
    acc += jnp.dot(a_ref[1], wp_ref[128:256, :], preferred_element_type=jnp.float32)
    acc += jnp.dot(a_ref[2], wp_ref[256:384, :], preferred_element_type=jnp.float32)
    acc += jnp.dot(a_ref[3], wp_ref[384:512, :], preferred_element_type=jnp.float32)
    p = jnp.maximum(acc + bp_ref[...], 0.0)
    q = jnp.maximum(
        jnp.dot(p, w3_ref[...], preferred_element_type=jnp.float32) + b3_ref[...], 0.0
    )
    o_ref[...] = jnp.dot(q, w4_ref[...], preferred_element_type=jnp.float32) + b4_ref[...]


def _head(a, Wp, bp, W3, b3, W4, b4):
    n = a.shape[1]
    full = lambda s: pl.BlockSpec(s, lambda i: (0,) * len(s))
    return pl.pallas_call(
        _head_body,
        grid=(n // BLK,),
        in_specs=[
            pl.BlockSpec((4, BLK, 128), lambda i: (0, i, 0)),
            full((512, 512)),
            full((1, 512)),
            full((512, 512)),
            full((1, 512)),
            full((512, 1)),
            full((1, 1)),
        ],
        out_specs=pl.BlockSpec((BLK, 1), lambda i: (i, 0)),
        out_shape=jax.ShapeDtypeStruct((n, 1), jnp.float32),
    )(a, Wp, bp.reshape(1, 512), W3, b3.reshape(1, 512), W4, b4.reshape(1, 1))


def _compact_body(src_hbm, dst_hbm, wl_hbm, stg, esrc, edst):
    wid = lax.axis_index("s") * 2 + lax.axis_index("c")
    iota16 = lax.iota(jnp.int32, 16)
    my_base = wid * PH_PER
    wl_base = wid * WL_CAP

    def chunk_body(k, carry):
        ptr, wtot = carry
        s16 = esrc[pl.ds(k * 16, 16)]
        d16 = edst[pl.ds(k * 16, 16)]
        u = d16 - my_base
        mask = (u >= 0) & (u < PH_PER)
        mi = jnp.where(mask, 1, 0).astype(jnp.int32)
        pos = ptr + plsc.cumsum(mi) - 1
        packed = lax.shift_left(s16, 10) | u
        plsc.store_scatter(stg, [pos], packed, mask=mask)
        ptr = ptr + jnp.sum(mi)

        def do_flush(p, w):
            pltpu.sync_copy(
                stg.at[pl.ds(0, FLUSH)],
                wl_hbm.at[pl.ds(pl.multiple_of(wl_base + 128 + w, 128), FLUSH)])
            rem = stg[pl.ds(FLUSH, 16)]
            stg[pl.ds(0, 16)] = rem
            return p - FLUSH, w + FLUSH

        ptr, wtot = lax.cond(ptr >= FLUSH, do_flush, lambda p, w: (p, w), ptr, wtot)
        return ptr, wtot

    def block_body(b, carry):
        pltpu.sync_copy(src_hbm.at[pl.ds(pl.multiple_of(b * EBLK, 128), EBLK)], esrc)
        pltpu.sync_copy(dst_hbm.at[pl.ds(pl.multiple_of(b * EBLK, 128), EBLK)], edst)
        return pl.loop(0, EBLK // 16, init_carry=carry)(chunk_body)

    ptr, wtot = pl.loop(0, E_PAD // EBLK,
                        init_carry=(jnp.int32(0), jnp.int32(0)))(block_body)

    sent_v = jnp.full((16,), SENT, jnp.int32)
    for t in range(ROUND // 16):
        plsc.store_scatter(stg, [ptr + iota16 + 16 * t], sent_v)
    ptrp = lax.div(ptr + (ROUND - 1), ROUND) * ROUND
    pltpu.sync_copy(
        stg.at[pl.ds(0, STG_CAP)],
        wl_hbm.at[pl.ds(pl.multiple_of(wl_base + 128 + wtot, 128), STG_CAP)])
    stg[pl.ds(0, 16)] = jnp.full((16,), 0, jnp.int32) + (wtot + ptrp)
    pltpu.sync_copy(stg.at[pl.ds(0, 128)],
                    wl_hbm.at[pl.ds(pl.multiple_of(wl_base, 128), 128)])


def _compact(src, dst):
    mesh = plsc.VectorSubcoreMesh(core_axis_name="c", subcore_axis_name="s")
    cp = pltpu.CompilerParams()
    if "needs_layout_passes" in pltpu.CompilerParams.__dataclass_fields__:
        cp = dataclasses.replace(cp, needs_layout_passes=False)
    return pl.kernel(
        _compact_body,
        out_type=[jax.ShapeDtypeStruct((NW * WL_CAP,), jnp.int32)],
        mesh=mesh,
        scratch_types=[
            pltpu.VMEM((STG_CAP,), jnp.int32),
            pltpu.VMEM((EBLK,), jnp.int32),
            pltpu.VMEM((EBLK,), jnp.int32),
        ],
        compiler_params=cp,
    )(src, dst)[0]


def _gmax_body(wl_hbm, h_hbm, a_hbm,
               agg_t, wlv, ibuf, gbuf,
               wsem0, wsem1, gsem0, gsem1):
    wid = lax.axis_index("s") * 2 + lax.axis_index("c")
    my_base = wid * PH_PER
    wl_base = wid * WL_CAP
    wsems = (wsem0, wsem1)
    gsems = (gsem0, gsem1)

    pltpu.sync_copy(wl_hbm.at[pl.ds(pl.multiple_of(wl_base, 128), 128)],
                    wlv.at[0])
    n_w = wlv[0, pl.ds(0, 16)][0]
    n_rounds = lax.div(n_w, ROUND)
    n_super = lax.div(n_rounds + 1, 2)

    def issue_gathers(f, wlb, half, gb):
        for g in range(4):
            pk = wlv[wlb, pl.ds(half * 64 + g * 16, 16)]
            s16 = lax.shift_right_logical(pk, 10) + f * N_PAD
            ibuf[gb, pl.ds(g * 16, 16)] = s16
        pltpu.async_copy(h_hbm.at[ibuf.at[gb]], gbuf.at[gb], gsems[gb])

    def drain_gathers(gb):
        pltpu.make_async_copy(
            h_hbm.at[pl.ds(0, 64)], gbuf.at[gb], gsems[gb]
        ).wait()

    def process_round(wlb, half, gb):
        @pl.loop(0, 4)
        def _(g):
            pk16 = wlv[wlb, pl.ds(half * 64 + g * 16, 16)]
            u16 = pk16 & 1023
            for e in range(16):
                u = u16[e]
                row = g * 16 + e
                for v in range(8):
                    gval = gbuf[gb, row, pl.ds(16 * v, 16)]
                    old = agg_t[u, pl.ds(16 * v, 16)]
                    agg_t[u, pl.ds(16 * v, 16)] = jnp.maximum(old, gval)

    def fetch_wl(j, wlb):
        off = pl.ds(pl.multiple_of(wl_base + 128 + j * WFETCH, 128), WFETCH)
        pltpu.async_copy(wl_hbm.at[off], wlv.at[wlb], wsems[wlb])

    def wait_wl(wlb):
        pltpu.make_async_copy(wl_hbm.at[pl.ds(0, WFETCH)], wlv.at[wlb],
                              wsems[wlb]).wait()

    z16 = jnp.zeros((16,), jnp.float32)

    @pl.loop(0, 4)
    def _(f):
        @pl.loop(0, AGG_ROWS)
        def _(r):
            for c in range(8):
                agg_t[r, pl.ds(c * 16, 16)] = z16

        @pl.when(n_rounds > 0)
        def _():
            pltpu.sync_copy(
                wl_hbm.at[pl.ds(pl.multiple_of(wl_base + 128, 128), WFETCH)],
                wlv.at[0])
            issue_gathers(f, 0, 0, 0)

        @pl.loop(0, n_super)
        def _(j):
            wlb = lax.rem(j, 2)
            r0 = j * 2

            def super_body(wlb, wlbn):
                @pl.when(j + 1 < n_super)
                def _():
                    fetch_wl(j + 1, wlbn)

                @pl.when(r0 + 1 < n_rounds)
                def _():
                    issue_gathers(f, wlb, 1, 1)

                drain_gathers(0)
                process_round(wlb, 0, 0)

                @pl.when(r0 + 1 < n_rounds)
                def _():
                    @pl.when(j + 1 < n_super)
                    def _():
                        wait_wl(wlbn)
                        issue_gathers(f, wlbn, 0, 0)

                    drain_gathers(1)
                    process_round(wlb, 1, 1)

            @pl.when(wlb == 0)
            def _():
                super_body(0, 1)

            @pl.when(wlb == 1)
            def _():
                super_body(1, 0)

        pltpu.sync_copy(
            agg_t.at[pl.ds(0, PH_PER)],
            a_hbm.at[pl.ds(pl.multiple_of(f * N_PAD + my_base, 8), PH_PER)])


def _gmax(wl, h):
    mesh = plsc.VectorSubcoreMesh(core_axis_name="c", subcore_axis_name="s")
    cp = pltpu.CompilerParams()
    if "needs_layout_passes" in pltpu.CompilerParams.__dataclass_fields__:
        cp = dataclasses.replace(cp, needs_layout_passes=False)
    return pl.kernel(
        _gmax_body,
        out_type=[jax.ShapeDtypeStruct((4 * N_PAD, 128), jnp.float32)],
        mesh=mesh,
        scratch_types=[
            pltpu.VMEM((AGG_ROWS, 128), jnp.float32),
            pltpu.VMEM((2, WFETCH), jnp.int32),
            pltpu.VMEM((2, 64), jnp.int32),
            pltpu.VMEM((2, 64, 128), jnp.float32),
            pltpu.SemaphoreType.DMA,
            pltpu.SemaphoreType.DMA,
            pltpu.SemaphoreType.DMA,
            pltpu.SemaphoreType.DMA,
        ],
        compiler_params=cp,
    )(wl, h)[0]


def kernel(x_movement, x_phase, edge_index_mp, W1, b1, W2, b2, Wp, bp, W3, b3, W4, b4):
    del x_phase
    src = jnp.pad(edge_index_mp[0], (0, E_PAD - E))
    dst = jnp.pad(edge_index_mp[1], (0, E_PAD - E), constant_values=1 << 20)
    wl = _compact(src, dst)
    xm = jnp.pad(x_movement[:N_PHASE], ((0, N_PAD - N_PHASE), (0, 0)))
    h = _mov_mlp(xm, W1, b1, W2, b2).reshape(4 * N_PAD, 128)
    a = _gmax(wl, h).reshape(4, N_PAD, 128)
    out = _head(a, Wp, bp, W3, b3, W4, b4)
    return out[:N_PHASE]


# speedup vs baseline: 2.0695x; 1.1998x over previous
import dataclasses

import jax
import jax.numpy as jnp
from jax import lax
from jax.experimental import pallas as pl
from jax.experimental.pallas import tpu as pltpu
from jax.experimental.pallas import tpu_sc as plsc

N_PHASE = 25000
N_PAD = 25088
BLK = 512
E = 100000
E_PAD = 100096

NW = 32
PH_PER = 784
AGG_ROWS = 788
SENT = PH_PER
EBLK = 4352
STG_CAP = 2048
FLUSH = 1920
WL_CAP = 102400
ROUND = 64
WFETCH = 128


def _mov_mlp_body(x_ref, w1_ref, b1_ref, w2_ref, b2_ref, h_ref):
    x = x_ref[...]
    h = jnp.maximum(
        jnp.dot(x, w1_ref[...], preferred_element_type=jnp.float32) + b1_ref[...], 0.0
    )
    h = jnp.maximum(
        jnp.dot(h, w2_ref[...], preferred_element_type=jnp.float32) + b2_ref[...], 0.0
    )
    hb = lax.bitcast_convert_type(h, jnp.uint32) + jnp.uint32(0x8000)
    lo = lax.shift_right_logical(hb, jnp.uint32(16))
    hi = hb & jnp.uint32(0xFFFF0000)
    blocks = [lo[:, 32 * k:32 * k + 16] | hi[:, 32 * k + 16:32 * k + 32]
              for k in range(16)]
    for p in range(2):
        h_ref[p] = lax.bitcast_convert_type(
            jnp.concatenate(blocks[8 * p:8 * p + 8], axis=1), jnp.int32)


def _mov_mlp(x, W1, b1, W2, b2):
    n = x.shape[0]
    full = lambda s: pl.BlockSpec(s, lambda i: (0,) * len(s))
    return pl.pallas_call(
        _mov_mlp_body,
        grid=(n // BLK,),
        in_specs=[
            pl.BlockSpec((BLK, 128), lambda i: (i, 0)),
            full((128, 512)),
            full((1, 512)),
            full((512, 512)),
            full((1, 512)),
        ],
        out_specs=pl.BlockSpec((2, BLK, 128), lambda i: (0, i, 0)),
        out_shape=jax.ShapeDtypeStruct((2, n, 128), jnp.int32),
    )(x, W1, b1.reshape(1, 512), W2, b2.reshape(1, 512))


def _head_body(a_ref, wp_ref, bp_ref, w3_ref, b3_ref, w4_ref, b4_ref, o_ref):
    parts = []
    for p in range(2):
        w = a_ref[p]
        lo = lax.bitcast_convert_type(lax.shift_left(w, 16), jnp.float32)
        hi = lax.bitcast_convert_type(w & jnp.int32(-65536), jnp.float32)
        for k in range(8):
            parts.append(lo[:, 16 * k:16 * k + 16])
            parts.append(hi[:, 16 * k:16 * k + 16])
    hf = jnp.concatenate(parts, axis=1)
    acc = jnp.dot(hf, wp_ref[...], preferred_element_type=jnp.float32)
    p = jnp.maximum(acc + bp_ref[...], 0.0)
    q = jnp.maximum(
        jnp.dot(p, w3_ref[...], preferred_element_type=jnp.float32) + b3_ref[...], 0.0
    )
    o_ref[...] = jnp.dot(q, w4_ref[...], preferred_element_type=jnp.float32) + b4_ref[...]


def _head(a, Wp, bp, W3, b3, W4, b4):
    n = a.shape[1]
    full = lambda s: pl.BlockSpec(s, lambda i: (0,) * len(s))
    return pl.pallas_call(
        _head_body,
        grid=(n // BLK,),
        in_specs=[
            pl.BlockSpec((2, BLK, 128), lambda i: (0, i, 0)),
            full((512, 512)),
            full((1, 512)),
            full((512, 512)),
            full((1, 512)),
            full((512, 1)),
            full((1, 1)),
        ],
        out_specs=pl.BlockSpec((BLK, 1), lambda i: (i, 0)),
        out_shape=jax.ShapeDtypeStruct((n, 1), jnp.float32),
    )(a, Wp, bp.reshape(1, 512), W3, b3.reshape(1, 512), W4, b4.reshape(1, 1))


def _compact_body(src_hbm, dst_hbm, wl_hbm, stg, esrc, edst):
    wid = lax.axis_index("s") * 2 + lax.axis_index("c")
    iota16 = lax.iota(jnp.int32, 16)
    my_base = wid * PH_PER
    wl_base = wid * WL_CAP

    def chunk_body(k, carry):
        ptr, wtot = carry
        s16 = esrc[pl.ds(k * 16, 16)]
        d16 = edst[pl.ds(k * 16, 16)]
        u = d16 - my_base
        mask = (u >= 0) & (u < PH_PER)
        mi = jnp.where(mask, 1, 0).astype(jnp.int32)
        pos = ptr + plsc.cumsum(mi) - 1
        packed = lax.shift_left(s16, 10) | u
        plsc.store_scatter(stg, [pos], packed, mask=mask)
        ptr = ptr + jnp.sum(mi)

        def do_flush(p, w):
            pltpu.sync_copy(
                stg.at[pl.ds(0, FLUSH)],
                wl_hbm.at[pl.ds(pl.multiple_of(wl_base + 128 + w, 128), FLUSH)])
            rem = stg[pl.ds(FLUSH, 16)]
            stg[pl.ds(0, 16)] = rem
            return p - FLUSH, w + FLUSH

        ptr, wtot = lax.cond(ptr >= FLUSH, do_flush, lambda p, w: (p, w), ptr, wtot)
        return ptr, wtot

    def block_body(b, carry):
        pltpu.sync_copy(src_hbm.at[pl.ds(pl.multiple_of(b * EBLK, 128), EBLK)], esrc)
        pltpu.sync_copy(dst_hbm.at[pl.ds(pl.multiple_of(b * EBLK, 128), EBLK)], edst)
        return pl.loop(0, EBLK // 16, init_carry=carry)(chunk_body)

    ptr, wtot = pl.loop(0, E_PAD // EBLK,
                        init_carry=(jnp.int32(0), jnp.int32(0)))(block_body)

    sent_v = jnp.full((16,), SENT, jnp.int32)
    for t in range(ROUND // 16):
        plsc.store_scatter(stg, [ptr + iota16 + 16 * t], sent_v)
    ptrp = lax.div(ptr + (ROUND - 1), ROUND) * ROUND
    pltpu.sync_copy(
        stg.at[pl.ds(0, STG_CAP)],
        wl_hbm.at[pl.ds(pl.multiple_of(wl_base + 128 + wtot, 128), STG_CAP)])
    stg[pl.ds(0, 16)] = jnp.full((16,), 0, jnp.int32) + (wtot + ptrp)
    pltpu.sync_copy(stg.at[pl.ds(0, 128)],
                    wl_hbm.at[pl.ds(pl.multiple_of(wl_base, 128), 128)])


def _compact(src, dst):
    mesh = plsc.VectorSubcoreMesh(core_axis_name="c", subcore_axis_name="s")
    cp = pltpu.CompilerParams()
    if "needs_layout_passes" in pltpu.CompilerParams.__dataclass_fields__:
        cp = dataclasses.replace(cp, needs_layout_passes=False)
    return pl.kernel(
        _compact_body,
        out_type=[jax.ShapeDtypeStruct((NW * WL_CAP,), jnp.int32)],
        mesh=mesh,
        scratch_types=[
            pltpu.VMEM((STG_CAP,), jnp.int32),
            pltpu.VMEM((EBLK,), jnp.int32),
            pltpu.VMEM((EBLK,), jnp.int32),
        ],
        compiler_params=cp,
    )(src, dst)[0]


def _gmax_body(wl_hbm, h_hbm, a_hbm,
               agg_t, wlv, ibuf, gbuf,
               wsem0, wsem1, gsem0, gsem1):
    wid = lax.axis_index("s") * 2 + lax.axis_index("c")
    my_base = wid * PH_PER
    wl_base = wid * WL_CAP
    wsems = (wsem0, wsem1)
    gsems = (gsem0, gsem1)

    pltpu.sync_copy(wl_hbm.at[pl.ds(pl.multiple_of(wl_base, 128), 128)],
                    wlv.at[0])
    n_w = wlv[0, pl.ds(0, 16)][0]
    n_rounds = lax.div(n_w, ROUND)
    n_super = lax.div(n_rounds + 1, 2)

    def issue_gathers(f, wlb, half, gb):
        for g in range(4):
            pk = wlv[wlb, pl.ds(half * 64 + g * 16, 16)]
            s16 = lax.shift_right_logical(pk, 10) + f * N_PAD
            ibuf[gb, pl.ds(g * 16, 16)] = s16
        pltpu.async_copy(h_hbm.at[ibuf.at[gb]], gbuf.at[gb], gsems[gb])

    def drain_gathers(gb):
        pltpu.make_async_copy(
            h_hbm.at[pl.ds(0, 64)], gbuf.at[gb], gsems[gb]
        ).wait()

    def process_round(wlb, half, gb):
        @pl.loop(0, 4)
        def _(g):
            pk16 = wlv[wlb, pl.ds(half * 64 + g * 16, 16)]
            u16 = pk16 & 1023
            for e in range(16):
                u = u16[e]
                row = g * 16 + e
                for wg in range(8):
                    w = gbuf[gb, row, pl.ds(16 * wg, 16)]
                    a = agg_t[u, pl.ds(16 * wg, 16)]
                    mlo = jnp.maximum(
                        plsc.bitcast(lax.shift_left(w, 16), jnp.float32),
                        plsc.bitcast(lax.shift_left(a, 16), jnp.float32))
                    mhi = jnp.maximum(
                        plsc.bitcast(w & jnp.int32(-65536), jnp.float32),
                        plsc.bitcast(a & jnp.int32(-65536), jnp.float32))
                    agg_t[u, pl.ds(16 * wg, 16)] = (
                        lax.shift_right_logical(plsc.bitcast(mlo, jnp.int32), 16)
                        | plsc.bitcast(mhi, jnp.int32))

    def fetch_wl(j, wlb):
        off = pl.ds(pl.multiple_of(wl_base + 128 + j * WFETCH, 128), WFETCH)
        pltpu.async_copy(wl_hbm.at[off], wlv.at[wlb], wsems[wlb])

    def wait_wl(wlb):
        pltpu.make_async_copy(wl_hbm.at[pl.ds(0, WFETCH)], wlv.at[wlb],
                              wsems[wlb]).wait()

    z16 = jnp.zeros((16,), jnp.int32)

    @pl.loop(0, 2)
    def _(f):
        @pl.loop(0, AGG_ROWS)
        def _(r):
            for c in range(8):
                agg_t[r, pl.ds(c * 16, 16)] = z16

        @pl.when(n_rounds > 0)
        def _():
            pltpu.sync_copy(
                wl_hbm.at[pl.ds(pl.multiple_of(wl_base + 128, 128), WFETCH)],
                wlv.at[0])
            issue_gathers(f, 0, 0, 0)

        @pl.loop(0, n_super)
        def _(j):
            wlb = lax.rem(j, 2)
            r0 = j * 2

            def super_body(wlb, wlbn):
                @pl.when(j + 1 < n_super)
                def _():
                    fetch_wl(j + 1, wlbn)

                @pl.when(r0 + 1 < n_rounds)
                def _():
                    issue_gathers(f, wlb, 1, 1)

                drain_gathers(0)
                process_round(wlb, 0, 0)

                @pl.when(r0 + 1 < n_rounds)
                def _():
                    @pl.when(j + 1 < n_super)
                    def _():
                        wait_wl(wlbn)
                        issue_gathers(f, wlbn, 0, 0)

                    drain_gathers(1)
                    process_round(wlb, 1, 1)

            @pl.when(wlb == 0)
            def _():
                super_body(0, 1)

            @pl.when(wlb == 1)
            def _():
                super_body(1, 0)

        pltpu.sync_copy(
            agg_t.at[pl.ds(0, PH_PER)],
            a_hbm.at[pl.ds(pl.multiple_of(f * N_PAD + my_base, 8), PH_PER)])


def _gmax(wl, h):
    mesh = plsc.VectorSubcoreMesh(core_axis_name="c", subcore_axis_name="s")
    cp = pltpu.CompilerParams()
    if "needs_layout_passes" in pltpu.CompilerParams.__dataclass_fields__:
        cp = dataclasses.replace(cp, needs_layout_passes=False)
    return pl.kernel(
        _gmax_body,
        out_type=[jax.ShapeDtypeStruct((2 * N_PAD, 128), jnp.int32)],
        mesh=mesh,
        scratch_types=[
            pltpu.VMEM((AGG_ROWS, 128), jnp.int32),
            pltpu.VMEM((2, WFETCH), jnp.int32),
            pltpu.VMEM((2, 64), jnp.int32),
            pltpu.VMEM((2, 64, 128), jnp.int32),
            pltpu.SemaphoreType.DMA,
            pltpu.SemaphoreType.DMA,
            pltpu.SemaphoreType.DMA,
            pltpu.SemaphoreType.DMA,
        ],
        compiler_params=cp,
    )(wl, h)[0]


def kernel(x_movement, x_phase, edge_index_mp, W1, b1, W2, b2, Wp, bp, W3, b3, W4, b4):
    del x_phase
    src = jnp.pad(edge_index_mp[0], (0, E_PAD - E))
    dst = jnp.pad(edge_index_mp[1], (0, E_PAD - E), constant_values=1 << 20)
    wl = _compact(src, dst)
    xm = jnp.pad(x_movement[:N_PHASE], ((0, N_PAD - N_PHASE), (0, 0)))
    h = _mov_mlp(xm, W1, b1, W2, b2).reshape(2 * N_PAD, 128)
    a = _gmax(wl, h).reshape(2, N_PAD, 128)
    out = _head(a, Wp, bp, W3, b3, W4, b4)
    return out[:N_PHASE]
